# Initial kernel scaffold; baseline (speedup 1.0000x reference)
import jax
import jax.numpy as jnp
from jax.experimental import pallas as pl


def kernel(x, edge_index, W1, b1, W2, b2):
    raise NotImplementedError("write your pallas kernel here")


# speedup vs baseline: 24.1337x; 24.1337x over previous
import functools

import jax
import jax.numpy as jnp
from jax import lax
from jax.experimental import pallas as pl
from jax.experimental.pallas import tpu as pltpu
from jax.experimental.pallas import tpu_sc as plsc

NC = 2
NS = 16
NW = NC * NS
CH = 128


def _sc_mesh():
    return plsc.VectorSubcoreMesh(core_axis_name="c", subcore_axis_name="s")


def _sc_degree(dst3, zeros_n, n_pad, n_chunks, rows_pt):
    @functools.partial(
        pl.kernel,
        out_type=jax.ShapeDtypeStruct((NC * n_pad,), jnp.float32),
        mesh=_sc_mesh(),
        compiler_params=pltpu.CompilerParams(use_tc_tiling_on_sc=False),
        scratch_types=[
            pltpu.VMEM((n_chunks, CH), jnp.int32),
            pltpu.VMEM((CH,), jnp.float32),
            pltpu.VMEM((rows_pt,), jnp.float32),
            pltpu.VMEM_SHARED((n_pad,), jnp.float32),
        ],
    )
    def k(dst_hbm, z_hbm, out_hbm, dst_v, ones_v, stage_v, deg_s):
        ci = lax.axis_index("c")
        si = lax.axis_index("s")
        wid = si * NC + ci
        for i in range(CH // 16):
            ones_v[pl.ds(i * 16, 16)] = jnp.ones((16,), jnp.float32)
        pltpu.sync_copy(z_hbm.at[pl.ds(si * rows_pt, rows_pt)], stage_v)
        pltpu.sync_copy(stage_v, deg_s.at[pl.ds(si * rows_pt, rows_pt)])
        pltpu.sync_copy(dst_hbm.at[wid], dst_v)
        plsc.subcore_barrier()

        def body(j, _):
            pltpu.sync_copy(ones_v, deg_s.at[dst_v.at[j]], add=True)
            return ()

        lax.fori_loop(0, n_chunks, body, ())
        plsc.subcore_barrier()
        pltpu.sync_copy(deg_s.at[pl.ds(si * rows_pt, rows_pt)], stage_v)
        pltpu.sync_copy(stage_v,
                        out_hbm.at[pl.ds(ci * n_pad + si * rows_pt, rows_pt)])

    return k(dst3, zeros_n).reshape(NC, n_pad)


def _sc_aggregate(src3, dst3, h, zeros_w, n_pad, n_chunks, rows_pt):
    W = h.shape[1]

    @functools.partial(
        pl.kernel,
        out_type=jax.ShapeDtypeStruct((NC, n_pad, W), jnp.float32),
        mesh=_sc_mesh(),
        compiler_params=pltpu.CompilerParams(use_tc_tiling_on_sc=False),
        scratch_types=[
            pltpu.VMEM((n_chunks, CH), jnp.int32),
            pltpu.VMEM((n_chunks, CH), jnp.int32),
            pltpu.VMEM((CH, W), jnp.float32),
            pltpu.VMEM((CH, W), jnp.float32),
            pltpu.VMEM((rows_pt, W), jnp.float32),
            pltpu.VMEM_SHARED((n_pad, W), jnp.float32),
            pltpu.SemaphoreType.DMA,
            pltpu.SemaphoreType.DMA,
        ],
    )
    def k(src_hbm, dst_hbm, h_hbm, z_hbm, out_hbm,
          src_v, dst_v, buf0, buf1, stage_v, acc_s, sem0, sem1):
        ci = lax.axis_index("c")
        si = lax.axis_index("s")
        wid = si * NC + ci
        pltpu.sync_copy(z_hbm.at[pl.ds(si * rows_pt, rows_pt)], stage_v)
        pltpu.sync_copy(stage_v, acc_s.at[pl.ds(si * rows_pt, rows_pt)])
        pltpu.sync_copy(src_hbm.at[wid], src_v)
        pltpu.sync_copy(dst_hbm.at[wid], dst_v)
        plsc.subcore_barrier()

        pltpu.async_copy(h_hbm.at[src_v.at[0]], buf0, sem0)
        pltpu.async_copy(h_hbm.at[src_v.at[1]], buf1, sem1)

        def body(i, _):
            j0 = 2 * i
            j1 = j0 + 1
            pltpu.make_async_copy(h_hbm.at[src_v.at[j0]], buf0, sem0).wait()
            pltpu.sync_copy(buf0, acc_s.at[dst_v.at[j0]], add=True)

            @pl.when(j0 + 2 < n_chunks)
            def _():
                pltpu.async_copy(h_hbm.at[src_v.at[j0 + 2]], buf0, sem0)

            pltpu.make_async_copy(h_hbm.at[src_v.at[j1]], buf1, sem1).wait()
            pltpu.sync_copy(buf1, acc_s.at[dst_v.at[j1]], add=True)

            @pl.when(j1 + 2 < n_chunks)
            def _():
                pltpu.async_copy(h_hbm.at[src_v.at[j1 + 2]], buf1, sem1)

            return ()

        lax.fori_loop(0, n_chunks // 2, body, ())
        plsc.subcore_barrier()
        pltpu.sync_copy(acc_s.at[pl.ds(si * rows_pt, rows_pt)], stage_v)
        pltpu.sync_copy(stage_v, out_hbm.at[ci, pl.ds(si * rows_pt, rows_pt)])

    return k(src3, dst3, h, zeros_w)


def _tc_layer1(degp, x, W1, rb):
    n, d_in = x.shape
    d_hid = W1.shape[1]

    def body(degp_ref, x_ref, w_ref, hs_ref, dis_ref):
        deg = degp_ref[0] + degp_ref[1] + 1.0
        dis = lax.rsqrt(jnp.maximum(deg, 1.0))
        hval = jnp.dot(x_ref[...], w_ref[...],
                       preferred_element_type=jnp.float32)
        hs_ref[...] = hval * dis
        dis_ref[...] = dis

    return pl.pallas_call(
        body,
        grid=(n // rb,),
        in_specs=[
            pl.BlockSpec((NC, rb, 1), lambda i: (0, i, 0)),
            pl.BlockSpec((rb, d_in), lambda i: (i, 0)),
            pl.BlockSpec((d_in, d_hid), lambda i: (0, 0)),
        ],
        out_specs=[
            pl.BlockSpec((rb, d_hid), lambda i: (i, 0)),
            pl.BlockSpec((rb, 1), lambda i: (i, 0)),
        ],
        out_shape=[
            jax.ShapeDtypeStruct((n, d_hid), jnp.float32),
            jax.ShapeDtypeStruct((n, 1), jnp.float32),
        ],
    )(degp, x, W1)


def _tc_layer2(aggp, hs, dis, W2p, b1, rb, wp):
    n, d_hid = hs.shape

    def body(aggp_ref, hs_ref, dis_ref, w_ref, b_ref, hs2_ref):
        agg = aggp_ref[0] + aggp_ref[1] + hs_ref[...]
        r = jnp.maximum(agg * dis_ref[...] + b_ref[...], 0.0)
        h2 = jnp.dot(r, w_ref[...], preferred_element_type=jnp.float32)
        hs2_ref[...] = h2 * dis_ref[...]

    return pl.pallas_call(
        body,
        grid=(n // rb,),
        in_specs=[
            pl.BlockSpec((NC, rb, d_hid), lambda i: (0, i, 0)),
            pl.BlockSpec((rb, d_hid), lambda i: (i, 0)),
            pl.BlockSpec((rb, 1), lambda i: (i, 0)),
            pl.BlockSpec((d_hid, wp), lambda i: (0, 0)),
            pl.BlockSpec((1, d_hid), lambda i: (0, 0)),
        ],
        out_specs=pl.BlockSpec((rb, wp), lambda i: (i, 0)),
        out_shape=jax.ShapeDtypeStruct((n, wp), jnp.float32),
    )(aggp, hs, dis, W2p, b1)


def _tc_final(agg2p, hs2w, dis, b2, rb, d_out):
    n, wp = hs2w.shape

    def body(ap_ref, hs2_ref, dis_ref, b_ref, out_ref):
        a = ap_ref[0] + ap_ref[1] + hs2_ref[...]
        o = (a * dis_ref[...])[:, :d_out] + b_ref[...]
        m = jnp.max(o, axis=1, keepdims=True)
        lse = m + jnp.log(jnp.sum(jnp.exp(o - m), axis=1, keepdims=True))
        out_ref[...] = o - lse

    return pl.pallas_call(
        body,
        grid=(n // rb,),
        in_specs=[
            pl.BlockSpec((NC, rb, wp), lambda i: (0, i, 0)),
            pl.BlockSpec((rb, wp), lambda i: (i, 0)),
            pl.BlockSpec((rb, 1), lambda i: (i, 0)),
            pl.BlockSpec((1, d_out), lambda i: (0, 0)),
        ],
        out_specs=pl.BlockSpec((rb, d_out), lambda i: (i, 0)),
        out_shape=jax.ShapeDtypeStruct((n, d_out), jnp.float32),
    )(agg2p, hs2w, dis, b2)


def kernel(x, edge_index, W1, b1, W2, b2):
    n = x.shape[0]
    e = edge_index.shape[1]
    d_hid = W1.shape[1]
    d_out = W2.shape[1]
    wp = 16

    n_chunks = -(-e // (NW * CH))
    n_chunks += n_chunks % 2
    e_pad = n_chunks * NW * CH
    n_pad = -(-(n + 1) // (NS * 8)) * (NS * 8)
    rows_pt = n_pad // NS
    rb = 1000 if n % 1000 == 0 else 8

    src = edge_index[0]
    dst = edge_index[1]
    src3 = jnp.concatenate(
        [src, jnp.zeros((e_pad - e,), jnp.int32)]).reshape(NW, n_chunks, CH)
    dst3 = jnp.concatenate(
        [dst, jnp.full((e_pad - e,), n, jnp.int32)]).reshape(NW, n_chunks, CH)

    zeros_n = jnp.zeros((n_pad,), jnp.float32)
    zeros_h = jnp.zeros((n_pad, d_hid), jnp.float32)
    zeros_w = jnp.zeros((n_pad, wp), jnp.float32)
    W2p = jnp.pad(W2, ((0, 0), (0, wp - d_out)))

    degp = _sc_degree(dst3, zeros_n, n_pad, n_chunks, rows_pt)
    degp3 = degp[:, :n, None]
    hs, dis = _tc_layer1(degp3, x, W1, rb)
    aggp = _sc_aggregate(src3, dst3, hs, zeros_h, n_pad, n_chunks, rows_pt)
    hs2w = _tc_layer2(aggp, hs, dis, W2p, b1.reshape(1, d_hid), rb, wp)
    agg2p = _sc_aggregate(src3, dst3, hs2w, zeros_w, n_pad, n_chunks, rows_pt)
    return _tc_final(agg2p, hs2w, dis, b2.reshape(1, d_out), rb, d_out)
